# Initial kernel scaffold; baseline (speedup 1.0000x reference)
import jax
import jax.numpy as jnp
from jax.experimental import pallas as pl


def kernel(x, edge_index, weight, att_src, att_dst, bias):
    raise NotImplementedError("write your pallas kernel here")


# speedup vs baseline: 113.2760x; 113.2760x over previous
import functools

import jax
import jax.numpy as jnp
from jax import lax
from jax.experimental import pallas as pl
from jax.experimental.pallas import tpu as pltpu
from jax.experimental.pallas import tpu_sc as plsc

N = 10000
E = 320000
IN_C = 128
OUT_C = 32
HEADS = 4

NC = 2
NS = 16
NW = NC * NS
EW = E // NW
K = 80
NCHUNK = EW // K
SCH = 25
NSUPER = NCHUNK // SCH
BR = 624
TAIL = N - NS * BR
G16 = K // 16


def _tc_project(x, w2, a):
    bn = 1000

    def body(x_ref, w_ref, a_ref, xw_ref, al_ref):
        xw = jnp.dot(x_ref[...], w_ref[...], preferred_element_type=jnp.float32)
        xw_ref[...] = xw
        al_ref[...] = jnp.dot(xw, a_ref[...], preferred_element_type=jnp.float32)

    return pl.pallas_call(
        body,
        grid=(N // bn,),
        in_specs=[
            pl.BlockSpec((bn, IN_C), lambda i: (i, 0)),
            pl.BlockSpec((IN_C, IN_C), lambda i: (0, 0)),
            pl.BlockSpec((IN_C, 8), lambda i: (0, 0)),
        ],
        out_specs=[
            pl.BlockSpec((bn, IN_C), lambda i: (i, 0)),
            pl.BlockSpec((bn, 8), lambda i: (i, 0)),
        ],
        out_shape=[
            jax.ShapeDtypeStruct((N, IN_C), jnp.float32),
            jax.ShapeDtypeStruct((N, 8), jnp.float32),
        ],
    )(x, w2, a)


def _tc_normalize(u, s, bias, expand):
    bn = 1000

    def body(u_ref, s_ref, b_ref, e_ref, o_ref):
        ssum = s_ref[0, :, 0:HEADS] + s_ref[1, :, 0:HEADS]
        r = 1.0 / (ssum + 1e-8)
        rexp = jnp.dot(r, e_ref[...], preferred_element_type=jnp.float32)
        o_ref[...] = (u_ref[0] + u_ref[1]) * rexp + b_ref[...]

    return pl.pallas_call(
        body,
        grid=(N // bn,),
        in_specs=[
            pl.BlockSpec((2, bn, IN_C), lambda i: (0, i, 0)),
            pl.BlockSpec((2, bn, 8), lambda i: (0, i, 0)),
            pl.BlockSpec((1, IN_C), lambda i: (0, 0)),
            pl.BlockSpec((HEADS, IN_C), lambda i: (0, 0)),
        ],
        out_specs=pl.BlockSpec((bn, IN_C), lambda i: (i, 0)),
        out_shape=jax.ShapeDtypeStruct((N, IN_C), jnp.float32),
    )(u, s, bias.reshape(1, IN_C), expand)


def _make_sc_edge():
    mesh = plsc.VectorSubcoreMesh(core_axis_name="c", subcore_axis_name="s")

    @functools.partial(
        pl.kernel,
        out_type=[
            jax.ShapeDtypeStruct((NC, N, IN_C), jnp.float32),
            jax.ShapeDtypeStruct((NC, N, 8), jnp.float32),
        ],
        mesh=mesh,
        compiler_params=pltpu.CompilerParams(
            needs_layout_passes=False, use_tc_tiling_on_sc=False),
        scratch_types=[
            pltpu.VMEM((2, SCH, K), jnp.int32),
            pltpu.VMEM((2, SCH, K), jnp.int32),
            pltpu.VMEM((K, IN_C), jnp.float32),
            pltpu.VMEM((K, IN_C), jnp.float32),
            pltpu.VMEM((K, 8), jnp.float32),
            pltpu.VMEM((K, 8), jnp.float32),
            pltpu.VMEM((K, 8), jnp.float32),
            pltpu.VMEM((K, 8), jnp.float32),
            pltpu.VMEM((K, 8), jnp.float32),
            pltpu.VMEM((K, 8), jnp.float32),
            pltpu.VMEM_SHARED((N, IN_C), jnp.float32),
            pltpu.VMEM_SHARED((N, 8), jnp.float32),
            pltpu.SemaphoreType.DMA,
            pltpu.SemaphoreType.DMA,
            pltpu.SemaphoreType.DMA,
        ],
    )
    def sc_edge(xw_hbm, alph_hbm, rows_hbm, cols_hbm, zu_hbm, zs_hbm,
                u_out, s_out,
                rows_v, cols_v, g_a, g_b, ar_a, ar_b, ac_a, ac_b, e_a, e_b,
                u_sh, s_sh, sem_a, sem_b, sem_i):
        core = lax.axis_index("c")
        sid = lax.axis_index("s")
        wid = sid * NC + core
        base = sid * BR

        pltpu.sync_copy(rows_hbm.at[wid, 0], rows_v.at[0])
        pltpu.sync_copy(cols_hbm.at[wid, 0], cols_v.at[0])
        pltpu.async_copy(rows_hbm.at[wid, 1], rows_v.at[1], sem_i)
        pltpu.async_copy(cols_hbm.at[wid, 1], cols_v.at[1], sem_i)

        pltpu.sync_copy(zu_hbm, u_sh.at[pl.ds(base, BR)])
        pltpu.sync_copy(zs_hbm, s_sh.at[pl.ds(base, BR)])

        @pl.when(sid == NS - 1)
        def _zero_tail():
            pltpu.sync_copy(zu_hbm.at[pl.ds(0, TAIL)],
                            u_sh.at[pl.ds(NS * BR, TAIL)])
            pltpu.sync_copy(zs_hbm.at[pl.ds(0, TAIL)],
                            s_sh.at[pl.ds(NS * BR, TAIL)])

        iota = lax.broadcasted_iota(jnp.int32, (16,), 0)
        zero16 = jnp.zeros((16,), jnp.float32)

        def zero_erows(k, carry):
            e_idx = 2 * k + iota // 8
            h_idx = iota % 8
            plsc.store_scatter(e_a, [e_idx, h_idx], zero16)
            plsc.store_scatter(e_b, [e_idx, h_idx], zero16)
            return carry

        lax.fori_loop(0, K // 2, zero_erows, 0)
        plsc.subcore_barrier()

        def idx_issue(s, p):
            pltpu.async_copy(rows_hbm.at[wid, s], rows_v.at[p], sem_i)
            pltpu.async_copy(cols_hbm.at[wid, s], cols_v.at[p], sem_i)

        def idx_wait():
            pltpu.make_async_copy(rows_hbm.at[wid, 0], rows_v.at[0],
                                  sem_i).wait()
            pltpu.make_async_copy(cols_hbm.at[wid, 0], cols_v.at[0],
                                  sem_i).wait()

        def issue_gather(c, gbuf, arbuf, acbuf, sem, first=False):
            s = c // SCH
            p = s % 2
            cc = c - s * SCH
            if not first:
                @pl.when(jnp.logical_and(c < NCHUNK, cc == 0))
                def _new_super():
                    idx_wait()

                @pl.when(jnp.logical_and(
                    jnp.logical_and(c < NCHUNK, cc == 1),
                    jnp.logical_and(s >= 1, s + 1 < NSUPER)))
                def _prefetch_super():
                    idx_issue(s + 1, 1 - p)

            @pl.when(c < NCHUNK)
            def _issue():
                pltpu.async_copy(xw_hbm.at[cols_v.at[p, cc]], gbuf, sem)
                pltpu.async_copy(alph_hbm.at[rows_v.at[p, cc]], arbuf, sem)
                pltpu.async_copy(alph_hbm.at[cols_v.at[p, cc]], acbuf, sem)

        def wait_gather(gbuf, arbuf, acbuf, sem):
            pltpu.make_async_copy(xw_hbm.at[cols_v.at[0, 0]], gbuf,
                                  sem).wait()
            pltpu.make_async_copy(alph_hbm.at[rows_v.at[0, 0]], arbuf,
                                  sem).wait()
            pltpu.make_async_copy(alph_hbm.at[cols_v.at[0, 0]], acbuf,
                                  sem).wait()

        def compute_chunk(gbuf, arbuf, acbuf, ebuf):
            def group(g, carry):
                eidx = g * 16 + iota
                wl = []
                for h in range(HEADS):
                    asv = plsc.load_gather(
                        arbuf, [eidx, jnp.full((16,), h, jnp.int32)])
                    adv = plsc.load_gather(
                        acbuf, [eidx, jnp.full((16,), 4 + h, jnp.int32)])
                    aa = asv + adv
                    aa = jnp.maximum(aa, 0.2 * aa)
                    wv = jnp.exp(aa)
                    plsc.store_scatter(
                        ebuf, [eidx, jnp.full((16,), h, jnp.int32)], wv)
                    wl.append(wv)
                for e in range(16):
                    er = g * 16 + e
                    for h in range(HEADS):
                        wbc = jnp.full((16,), wl[h][e])
                        for jj in range(2):
                            c0 = h * OUT_C + jj * 16
                            gbuf[er, pl.ds(c0, 16)] = (
                                gbuf[er, pl.ds(c0, 16)] * wbc)
                return carry

            lax.fori_loop(0, G16, group, 0)

        def scatter_chunk(c, gbuf, ebuf):
            s = c // SCH
            p = s % 2
            cc = c - s * SCH
            pltpu.sync_copy(gbuf, u_sh.at[rows_v.at[p, cc]], add=True)
            pltpu.sync_copy(ebuf, s_sh.at[rows_v.at[p, cc]], add=True)

        issue_gather(0, g_a, ar_a, ac_a, sem_a, first=True)

        def pair(i, carry):
            c0 = i * 2
            c1 = c0 + 1
            wait_gather(g_a, ar_a, ac_a, sem_a)
            issue_gather(c1, g_b, ar_b, ac_b, sem_b)
            compute_chunk(g_a, ar_a, ac_a, e_a)
            scatter_chunk(c0, g_a, e_a)
            wait_gather(g_b, ar_b, ac_b, sem_b)
            issue_gather(c0 + 2, g_a, ar_a, ac_a, sem_a)
            compute_chunk(g_b, ar_b, ac_b, e_b)
            scatter_chunk(c1, g_b, e_b)
            return carry

        lax.fori_loop(0, (NCHUNK - 1) // 2, pair, 0)

        c_last = NCHUNK - 1
        wait_gather(g_a, ar_a, ac_a, sem_a)
        compute_chunk(g_a, ar_a, ac_a, e_a)
        scatter_chunk(c_last, g_a, e_a)

        plsc.subcore_barrier()
        pltpu.sync_copy(u_sh.at[pl.ds(base, BR)],
                        u_out.at[core, pl.ds(base, BR)])
        pltpu.sync_copy(s_sh.at[pl.ds(base, BR)],
                        s_out.at[core, pl.ds(base, BR)])

        @pl.when(sid == NS - 1)
        def _write_tail():
            pltpu.sync_copy(u_sh.at[pl.ds(NS * BR, TAIL)],
                            u_out.at[core, pl.ds(NS * BR, TAIL)])
            pltpu.sync_copy(s_sh.at[pl.ds(NS * BR, TAIL)],
                            s_out.at[core, pl.ds(NS * BR, TAIL)])

    return sc_edge


_sc_edge = _make_sc_edge()


def kernel(x, edge_index, weight, att_src, att_dst, bias):
    x = x.astype(jnp.float32)
    eye4 = jnp.eye(HEADS, dtype=jnp.float32)
    a_src = (att_src[0][:, :, None] * eye4[:, None, :]).reshape(IN_C, HEADS)
    a_dst = (att_dst[0][:, :, None] * eye4[:, None, :]).reshape(IN_C, HEADS)
    a_mat = jnp.concatenate([a_src, a_dst], axis=1)

    xw, alph = _tc_project(x, weight, a_mat)

    row = edge_index[0].astype(jnp.int32).reshape(NW, NSUPER, SCH, K)
    col = edge_index[1].astype(jnp.int32).reshape(NW, NSUPER, SCH, K)
    zu = jnp.zeros((BR, IN_C), jnp.float32)
    zs = jnp.zeros((BR, 8), jnp.float32)

    u, s = _sc_edge(xw, alph, row, col, zu, zs)

    expand = jnp.repeat(eye4, OUT_C, axis=1)
    return _tc_normalize(u, s, bias, expand)
